# baseline (speedup 1.0000x reference)
import functools

import jax
import jax.numpy as jnp
from jax import lax
from jax.experimental import pallas as pl
from jax.experimental.pallas import tpu as pltpu
from jax.experimental.pallas import tpu_sc as plsc

N = 10000
E = 320000
H = 128
PW = 16
AW = H + PW

NC = 2
NS = 16
NW = NC * NS
K = 128
NBLK = 80
EPW = K * NBLK
EPAD = NW * EPW
NB0 = 50
NB1 = 110

NPAD = 10240
ROWS_PER_SUB = NPAD // NS
ZCH = 128

EBLK = 2048
NBLK_TC = 1000


def _silu(v):
    return v * jax.nn.sigmoid(v)


def _pre_body(x_ref, posp_ref, w1a_ref, w1b_ref, b1_ref, a_ref, b_ref):
    xb = x_ref[...]
    pp = posp_ref[...]
    a_ref[:, :H] = xb @ w1a_ref[...] + b1_ref[...]
    a_ref[:, H:] = pp
    b_ref[:, :H] = xb @ w1b_ref[...]
    b_ref[:, H:] = -pp


def _build_tables(x, posp, w1a, w1b, b1):
    return pl.pallas_call(
        _pre_body,
        grid=(N // NBLK_TC,),
        in_specs=[
            pl.BlockSpec((NBLK_TC, H), lambda i: (i, 0)),
            pl.BlockSpec((NBLK_TC, PW), lambda i: (i, 0)),
            pl.BlockSpec((H, H), lambda i: (0, 0)),
            pl.BlockSpec((H, H), lambda i: (0, 0)),
            pl.BlockSpec((1, H), lambda i: (0, 0)),
        ],
        out_specs=[
            pl.BlockSpec((NBLK_TC, AW), lambda i: (i, 0)),
            pl.BlockSpec((NBLK_TC, AW), lambda i: (i, 0)),
        ],
        out_shape=[
            jax.ShapeDtypeStruct((N, AW), jnp.float32),
            jax.ShapeDtypeStruct((N, AW), jnp.float32),
        ],
    )(x, posp, w1a, w1b, b1)


_MESH = plsc.VectorSubcoreMesh(
    core_axis_name="c", subcore_axis_name="s", num_cores=NC, num_subcores=NS
)


@functools.partial(
    pl.kernel,
    out_type=[
        jax.ShapeDtypeStruct((EPAD, H), jnp.float32),
        jax.ShapeDtypeStruct((EPAD, PW), jnp.float32),
    ],
    mesh=_MESH,
    scratch_types=[
        pltpu.VMEM((K,), jnp.int32),
        pltpu.VMEM((K,), jnp.int32),
        pltpu.VMEM((K,), jnp.int32),
        pltpu.VMEM((K,), jnp.int32),
        pltpu.VMEM((K, AW), jnp.float32),
        pltpu.VMEM((K, AW), jnp.float32),
        pltpu.VMEM((K, AW), jnp.float32),
        pltpu.VMEM((K, AW), jnp.float32),
        pltpu.SemaphoreType.DMA,
        pltpu.SemaphoreType.DMA,
        pltpu.SemaphoreType.DMA,
        pltpu.SemaphoreType.DMA,
        pltpu.SemaphoreType.DMA,
        pltpu.SemaphoreType.DMA,
    ],
    compiler_params=pltpu.CompilerParams(use_tc_tiling_on_sc=False),
)
def _sc_gather(a_hbm, b_hbm, send_hbm, rec_hbm, g_hbm, d_hbm,
               s0, s1, r0, r1, ab0, ab1, bb0, bb1,
               si0, si1, sg0, sg1, sw0, sw1):
    cid = lax.axis_index("c")
    sid = lax.axis_index("s")
    nblk = jnp.where(cid == 0, NB0, NB1)
    base0 = jnp.where(cid == 0, sid * (K * NB0),
                      NS * K * NB0 + sid * (K * NB1))
    sidx = [s0, s1]
    ridx = [r0, r1]
    abuf = [ab0, ab1]
    bbuf = [bb0, bb1]
    isem = [si0, si1]
    gsem = [sg0, sg1]
    wsem = [sw0, sw1]

    def fire_idx(b, blk):
        base = base0 + blk * K
        pltpu.async_copy(send_hbm.at[pl.ds(base, K)], sidx[b], isem[b])
        pltpu.async_copy(rec_hbm.at[pl.ds(base, K)], ridx[b], isem[b])

    def wait_idx(b):
        pltpu.make_async_copy(send_hbm.at[pl.ds(0, K)], sidx[b], isem[b]).wait()
        pltpu.make_async_copy(rec_hbm.at[pl.ds(0, K)], ridx[b], isem[b]).wait()

    def fire_gather(b):
        pltpu.async_copy(a_hbm.at[sidx[b]], abuf[b], gsem[b])
        pltpu.async_copy(b_hbm.at[ridx[b]], bbuf[b], gsem[b])

    def wait_gather(b):
        pltpu.make_async_copy(a_hbm.at[sidx[b]], abuf[b], gsem[b]).wait()
        pltpu.make_async_copy(b_hbm.at[ridx[b]], bbuf[b], gsem[b]).wait()

    def fire_write(b, blk):
        base = base0 + blk * K
        pltpu.async_copy(abuf[b].at[:, pl.ds(0, H)],
                         g_hbm.at[pl.ds(base, K)], wsem[b])
        pltpu.async_copy(abuf[b].at[:, pl.ds(H, PW)],
                         d_hbm.at[pl.ds(base, K)], wsem[b])

    def wait_write(b):
        pltpu.make_async_copy(abuf[b].at[:, pl.ds(0, H)],
                              g_hbm.at[pl.ds(0, K)], wsem[b]).wait()
        pltpu.make_async_copy(abuf[b].at[:, pl.ds(H, PW)],
                              d_hbm.at[pl.ds(0, K)], wsem[b]).wait()

    def compute(b):
        ab = abuf[b]
        bb = bbuf[b]

        @plsc.parallel_loop(0, K, unroll=2)
        def row(j):
            for q in range(AW // 16):
                sl = pl.ds(q * 16, 16)
                ab[j, sl] = ab[j, sl] + bb[j, sl]

    fire_idx(0, 0)
    wait_idx(0)
    fire_gather(0)
    fire_idx(1, 1)

    def pair(ii, carry):
        for b in range(2):
            i = ii * 2 + b
            nb = 1 - b
            wait_gather(b)

            @pl.when(i + 2 < nblk)
            def _():
                fire_idx(b, i + 2)

            @pl.when(i + 1 < nblk)
            def _():
                wait_idx(nb)

                @pl.when(i >= 1)
                def _():
                    wait_write(nb)

                fire_gather(nb)

            compute(b)
            fire_write(b, i)
        return carry

    lax.fori_loop(0, nblk // 2, pair, 0)
    wait_write(0)
    wait_write(1)


def _edge_body(g_ref, d_ref, w2_ref, wd_ref, b2_ref, m_ref):
    d = d_ref[...]
    dist = jnp.sqrt(jnp.sum(d * d, axis=1, keepdims=True))
    h = _silu(g_ref[...] + dist * wd_ref[...])
    m_ref[...] = _silu(h @ w2_ref[...] + b2_ref[...])


def _edge_mlp(g, dmat, w2, w1d, b2):
    return pl.pallas_call(
        _edge_body,
        grid=(EPAD // EBLK,),
        in_specs=[
            pl.BlockSpec((EBLK, H), lambda i: (i, 0)),
            pl.BlockSpec((EBLK, PW), lambda i: (i, 0)),
            pl.BlockSpec((H, H), lambda i: (0, 0)),
            pl.BlockSpec((1, H), lambda i: (0, 0)),
            pl.BlockSpec((1, H), lambda i: (0, 0)),
        ],
        out_specs=pl.BlockSpec((EBLK, H), lambda i: (i, 0)),
        out_shape=jax.ShapeDtypeStruct((EPAD, H), jnp.float32),
    )(g, dmat, w2, w1d, b2)


@functools.partial(
    pl.kernel,
    out_type=jax.ShapeDtypeStruct((NC, NPAD, H), jnp.float32),
    mesh=_MESH,
    scratch_types=[
        pltpu.VMEM((K,), jnp.int32),
        pltpu.VMEM((K,), jnp.int32),
        pltpu.VMEM((K, H), jnp.float32),
        pltpu.VMEM((K, H), jnp.float32),
        pltpu.VMEM_SHARED((NPAD, H), jnp.float32),
        pltpu.SemaphoreType.DMA,
        pltpu.SemaphoreType.DMA,
        pltpu.SemaphoreType.DMA,
        pltpu.SemaphoreType.DMA,
    ],
    compiler_params=pltpu.CompilerParams(use_tc_tiling_on_sc=False),
)
def _sc_scatter(m_hbm, rec_hbm, out_hbm, r0, r1, mb0, mb1, aggr,
                sl0, sl1, sa0, sa1):
    cid = lax.axis_index("c")
    sid = lax.axis_index("s")
    wid = sid * NC + cid
    base0 = wid * EPW
    ridx = [r0, r1]
    mbuf = [mb0, mb1]
    lsem = [sl0, sl1]
    asem = [sa0, sa1]

    def zrow(j, c2):
        for q in range(H // 16):
            mb0[j, pl.ds(q * 16, 16)] = jnp.zeros((16,), jnp.float32)
        return c2

    lax.fori_loop(0, ZCH, zrow, 0)

    def zchunk(t, c2):
        pltpu.sync_copy(mb0, aggr.at[pl.ds(sid * ROWS_PER_SUB + t * ZCH, ZCH)])
        return c2

    lax.fori_loop(0, ROWS_PER_SUB // ZCH, zchunk, 0)
    plsc.subcore_barrier()

    def fire_load(b, blk):
        base = base0 + blk * K
        pltpu.async_copy(rec_hbm.at[pl.ds(base, K)], ridx[b], lsem[b])
        pltpu.async_copy(m_hbm.at[pl.ds(base, K)], mbuf[b], lsem[b])

    def wait_load(b):
        pltpu.make_async_copy(rec_hbm.at[pl.ds(0, K)], ridx[b], lsem[b]).wait()
        pltpu.make_async_copy(m_hbm.at[pl.ds(0, K)], mbuf[b], lsem[b]).wait()

    def fire_add(b):
        pltpu.async_copy(mbuf[b], aggr.at[ridx[b]], asem[b], add=True)

    def wait_add(b):
        pltpu.make_async_copy(mbuf[b], aggr.at[ridx[b]], asem[b]).wait()

    fire_load(0, 0)
    fire_load(1, 1)

    def pair(ii, carry):
        for b in range(2):
            i = ii * 2 + b
            wait_load(b)
            fire_add(b)

            @pl.when(i + 2 < NBLK)
            def _():
                wait_add(b)
                fire_load(b, i + 2)

        return carry

    lax.fori_loop(0, NBLK // 2, pair, 0)
    wait_add(0)
    wait_add(1)
    plsc.subcore_barrier()

    def dchunk(t, c2):
        row0 = sid * ROWS_PER_SUB + t * ZCH
        pltpu.sync_copy(aggr.at[pl.ds(row0, ZCH)], mb0)
        pltpu.sync_copy(mb0, out_hbm.at[cid, pl.ds(row0, ZCH)])
        return c2

    lax.fori_loop(0, ROWS_PER_SUB // ZCH, dchunk, 0)


def _fin_body(x_ref, p0_ref, p1_ref, u1a_ref, u1b_ref, ub1_ref, u2_ref,
              ub2_ref, out_ref):
    aggr = p0_ref[0] + p1_ref[0]
    t = x_ref[...] @ u1a_ref[...] + aggr @ u1b_ref[...] + ub1_ref[...]
    out_ref[...] = _silu(t) @ u2_ref[...] + ub2_ref[...]


def _final_mlp(x, partials, u1a, u1b, ub1, u2, ub2):
    return pl.pallas_call(
        _fin_body,
        grid=(N // NBLK_TC,),
        in_specs=[
            pl.BlockSpec((NBLK_TC, H), lambda i: (i, 0)),
            pl.BlockSpec((1, NBLK_TC, H), lambda i: (0, i, 0)),
            pl.BlockSpec((1, NBLK_TC, H), lambda i: (1, i, 0)),
            pl.BlockSpec((H, H), lambda i: (0, 0)),
            pl.BlockSpec((H, H), lambda i: (0, 0)),
            pl.BlockSpec((1, H), lambda i: (0, 0)),
            pl.BlockSpec((H, H), lambda i: (0, 0)),
            pl.BlockSpec((1, H), lambda i: (0, 0)),
        ],
        out_specs=pl.BlockSpec((NBLK_TC, H), lambda i: (i, 0)),
        out_shape=jax.ShapeDtypeStruct((N, H), jnp.float32),
    )(x, partials, partials, u1a, u1b, ub1, u2, ub2)


def kernel(x, pos, edge_index, W1, b1, W2, b2, U1, ub1, U2, ub2):
    send = jnp.pad(edge_index[0], (0, EPAD - E))
    rec_g = jnp.pad(edge_index[1], (0, EPAD - E))
    rec_s = jnp.pad(edge_index[1], (0, EPAD - E), constant_values=N)
    posp = jnp.pad(pos, ((0, 0), (0, PW - 3)))
    w1a = W1[:H]
    w1b = W1[H:2 * H]
    w1d = W1[2 * H:2 * H + 1]

    a_tab, b_tab = _build_tables(x, posp, w1a, w1b, b1[None, :])
    g, dmat = _sc_gather(a_tab, b_tab, send, rec_g)
    m = _edge_mlp(g, dmat, W2, w1d, b2[None, :])
    partials = _sc_scatter(m, rec_s)
    return _final_mlp(x, partials, U1[:H], U1[H:], ub1[None, :], U2,
                      ub2[None, :])


# speedup vs baseline: 1.0185x; 1.0185x over previous
import functools

import jax
import jax.numpy as jnp
from jax import lax
from jax.experimental import pallas as pl
from jax.experimental.pallas import tpu as pltpu
from jax.experimental.pallas import tpu_sc as plsc

N = 10000
E = 320000
H = 128
PW = 16
AW = H + PW

NC = 2
NS = 16
NW = NC * NS
K = 128
NBLK = 80
EPW = K * NBLK
EPAD = NW * EPW
NB0 = 50
NB1 = 110

NPAD = 10240
ROWS_PER_SUB = NPAD // NS
ZCH = 128

EBLK = 2048
NBLK_TC = 1000


def _silu(v):
    return v * jax.nn.sigmoid(v)


def _pre_body(x_ref, posp_ref, w1a_ref, w1b_ref, b1_ref, a_ref, b_ref):
    xb = x_ref[...]
    pp = posp_ref[...]
    a_ref[:, :H] = xb @ w1a_ref[...] + b1_ref[...]
    a_ref[:, H:] = pp
    b_ref[:, :H] = xb @ w1b_ref[...]
    b_ref[:, H:] = -pp


def _build_tables(x, posp, w1a, w1b, b1):
    return pl.pallas_call(
        _pre_body,
        grid=(N // NBLK_TC,),
        in_specs=[
            pl.BlockSpec((NBLK_TC, H), lambda i: (i, 0)),
            pl.BlockSpec((NBLK_TC, PW), lambda i: (i, 0)),
            pl.BlockSpec((H, H), lambda i: (0, 0)),
            pl.BlockSpec((H, H), lambda i: (0, 0)),
            pl.BlockSpec((1, H), lambda i: (0, 0)),
        ],
        out_specs=[
            pl.BlockSpec((NBLK_TC, AW), lambda i: (i, 0)),
            pl.BlockSpec((NBLK_TC, AW), lambda i: (i, 0)),
        ],
        out_shape=[
            jax.ShapeDtypeStruct((N, AW), jnp.float32),
            jax.ShapeDtypeStruct((N, AW), jnp.float32),
        ],
    )(x, posp, w1a, w1b, b1)


_MESH = plsc.VectorSubcoreMesh(
    core_axis_name="c", subcore_axis_name="s", num_cores=NC, num_subcores=NS
)


@functools.partial(
    pl.kernel,
    out_type=[
        jax.ShapeDtypeStruct((EPAD, H), jnp.float32),
        jax.ShapeDtypeStruct((EPAD, PW), jnp.float32),
    ],
    mesh=_MESH,
    scratch_types=[
        pltpu.VMEM((K,), jnp.int32),
        pltpu.VMEM((K,), jnp.int32),
        pltpu.VMEM((K,), jnp.int32),
        pltpu.VMEM((K,), jnp.int32),
        pltpu.VMEM((K, AW), jnp.float32),
        pltpu.VMEM((K, AW), jnp.float32),
        pltpu.VMEM((K, AW), jnp.float32),
        pltpu.VMEM((K, AW), jnp.float32),
        pltpu.VMEM((K, H), jnp.float32),
        pltpu.VMEM((K, H), jnp.float32),
        pltpu.VMEM((K, PW), jnp.float32),
        pltpu.VMEM((K, PW), jnp.float32),
        pltpu.SemaphoreType.DMA,
        pltpu.SemaphoreType.DMA,
        pltpu.SemaphoreType.DMA,
        pltpu.SemaphoreType.DMA,
        pltpu.SemaphoreType.DMA,
        pltpu.SemaphoreType.DMA,
    ],
    compiler_params=pltpu.CompilerParams(use_tc_tiling_on_sc=False),
)
def _sc_gather(a_hbm, b_hbm, send_hbm, rec_hbm, g_hbm, d_hbm,
               s0, s1, r0, r1, ab0, ab1, bb0, bb1, gb0, gb1, db0, db1,
               si0, si1, sg0, sg1, sw0, sw1):
    cid = lax.axis_index("c")
    sid = lax.axis_index("s")
    wid = sid * NC + cid
    base0 = wid * EPW
    nblk = NBLK
    sidx = [s0, s1]
    ridx = [r0, r1]
    abuf = [ab0, ab1]
    bbuf = [bb0, bb1]
    gbuf = [gb0, gb1]
    dbuf = [db0, db1]
    isem = [si0, si1]
    gsem = [sg0, sg1]
    wsem = [sw0, sw1]

    def fire_idx(b, blk):
        base = base0 + blk * K
        pltpu.async_copy(send_hbm.at[pl.ds(base, K)], sidx[b], isem[b])
        pltpu.async_copy(rec_hbm.at[pl.ds(base, K)], ridx[b], isem[b])

    def wait_idx(b):
        pltpu.make_async_copy(send_hbm.at[pl.ds(0, K)], sidx[b], isem[b]).wait()
        pltpu.make_async_copy(rec_hbm.at[pl.ds(0, K)], ridx[b], isem[b]).wait()

    def fire_gather(b):
        pltpu.async_copy(a_hbm.at[sidx[b]], abuf[b], gsem[b])
        pltpu.async_copy(b_hbm.at[ridx[b]], bbuf[b], gsem[b])

    def wait_gather(b):
        pltpu.make_async_copy(a_hbm.at[sidx[b]], abuf[b], gsem[b]).wait()
        pltpu.make_async_copy(b_hbm.at[ridx[b]], bbuf[b], gsem[b]).wait()

    def fire_write(b, blk):
        base = base0 + blk * K
        pltpu.async_copy(gbuf[b], g_hbm.at[pl.ds(base, K)], wsem[b])
        pltpu.async_copy(dbuf[b], d_hbm.at[pl.ds(base, K)], wsem[b])

    def wait_write(b):
        pltpu.make_async_copy(gbuf[b], g_hbm.at[pl.ds(0, K)], wsem[b]).wait()
        pltpu.make_async_copy(dbuf[b], d_hbm.at[pl.ds(0, K)], wsem[b]).wait()

    def compute(b):
        ab = abuf[b]
        bb = bbuf[b]
        gb = gbuf[b]
        db = dbuf[b]

        @plsc.parallel_loop(0, K, unroll=2)
        def row(j):
            for q in range(H // 16):
                sl = pl.ds(q * 16, 16)
                gb[j, sl] = ab[j, sl] + bb[j, sl]
            db[j, :] = ab[j, pl.ds(H, PW)] + bb[j, pl.ds(H, PW)]

    fire_idx(0, 0)
    wait_idx(0)
    fire_gather(0)
    fire_idx(1, 1)

    def pair(ii, carry):
        for b in range(2):
            i = ii * 2 + b
            nb = 1 - b
            wait_gather(b)

            @pl.when(i + 2 < nblk)
            def _():
                fire_idx(b, i + 2)

            @pl.when(i + 1 < nblk)
            def _():
                wait_idx(nb)
                fire_gather(nb)

            @pl.when(i >= 2)
            def _():
                wait_write(b)

            compute(b)
            fire_write(b, i)
        return carry

    lax.fori_loop(0, nblk // 2, pair, 0)
    wait_write(0)
    wait_write(1)


def _edge_body(g_ref, d_ref, w2_ref, wd_ref, b2_ref, m_ref):
    d = d_ref[...]
    dist = jnp.sqrt(jnp.sum(d * d, axis=1, keepdims=True))
    h = _silu(g_ref[...] + dist * wd_ref[...])
    m_ref[...] = _silu(h @ w2_ref[...] + b2_ref[...])


def _edge_mlp(g, dmat, w2, w1d, b2):
    return pl.pallas_call(
        _edge_body,
        grid=(EPAD // EBLK,),
        in_specs=[
            pl.BlockSpec((EBLK, H), lambda i: (i, 0)),
            pl.BlockSpec((EBLK, PW), lambda i: (i, 0)),
            pl.BlockSpec((H, H), lambda i: (0, 0)),
            pl.BlockSpec((1, H), lambda i: (0, 0)),
            pl.BlockSpec((1, H), lambda i: (0, 0)),
        ],
        out_specs=pl.BlockSpec((EBLK, H), lambda i: (i, 0)),
        out_shape=jax.ShapeDtypeStruct((EPAD, H), jnp.float32),
    )(g, dmat, w2, w1d, b2)


@functools.partial(
    pl.kernel,
    out_type=jax.ShapeDtypeStruct((NC, NPAD, H), jnp.float32),
    mesh=_MESH,
    scratch_types=[
        pltpu.VMEM((K,), jnp.int32),
        pltpu.VMEM((K,), jnp.int32),
        pltpu.VMEM((K, H), jnp.float32),
        pltpu.VMEM((K, H), jnp.float32),
        pltpu.VMEM_SHARED((NPAD, H), jnp.float32),
        pltpu.SemaphoreType.DMA,
        pltpu.SemaphoreType.DMA,
        pltpu.SemaphoreType.DMA,
        pltpu.SemaphoreType.DMA,
    ],
    compiler_params=pltpu.CompilerParams(use_tc_tiling_on_sc=False),
)
def _sc_scatter(m_hbm, rec_hbm, out_hbm, r0, r1, mb0, mb1, aggr,
                sl0, sl1, sa0, sa1):
    cid = lax.axis_index("c")
    sid = lax.axis_index("s")
    wid = sid * NC + cid
    base0 = wid * EPW
    ridx = [r0, r1]
    mbuf = [mb0, mb1]
    lsem = [sl0, sl1]
    asem = [sa0, sa1]

    def zrow(j, c2):
        for q in range(H // 16):
            mb0[j, pl.ds(q * 16, 16)] = jnp.zeros((16,), jnp.float32)
        return c2

    lax.fori_loop(0, ZCH, zrow, 0)

    def zchunk(t, c2):
        pltpu.sync_copy(mb0, aggr.at[pl.ds(sid * ROWS_PER_SUB + t * ZCH, ZCH)])
        return c2

    lax.fori_loop(0, ROWS_PER_SUB // ZCH, zchunk, 0)
    plsc.subcore_barrier()

    def fire_load(b, blk):
        base = base0 + blk * K
        pltpu.async_copy(rec_hbm.at[pl.ds(base, K)], ridx[b], lsem[b])
        pltpu.async_copy(m_hbm.at[pl.ds(base, K)], mbuf[b], lsem[b])

    def wait_load(b):
        pltpu.make_async_copy(rec_hbm.at[pl.ds(0, K)], ridx[b], lsem[b]).wait()
        pltpu.make_async_copy(m_hbm.at[pl.ds(0, K)], mbuf[b], lsem[b]).wait()

    def fire_add(b):
        pltpu.async_copy(mbuf[b], aggr.at[ridx[b]], asem[b], add=True)

    def wait_add(b):
        pltpu.make_async_copy(mbuf[b], aggr.at[ridx[b]], asem[b]).wait()

    fire_load(0, 0)
    fire_load(1, 1)

    def pair(ii, carry):
        for b in range(2):
            i = ii * 2 + b
            wait_load(b)
            fire_add(b)

            @pl.when(i + 2 < NBLK)
            def _():
                wait_add(b)
                fire_load(b, i + 2)

        return carry

    lax.fori_loop(0, NBLK // 2, pair, 0)
    wait_add(0)
    wait_add(1)
    plsc.subcore_barrier()

    def dchunk(t, c2):
        row0 = sid * ROWS_PER_SUB + t * ZCH
        pltpu.sync_copy(aggr.at[pl.ds(row0, ZCH)], mb0)
        pltpu.sync_copy(mb0, out_hbm.at[cid, pl.ds(row0, ZCH)])
        return c2

    lax.fori_loop(0, ROWS_PER_SUB // ZCH, dchunk, 0)


def _fin_body(x_ref, p0_ref, p1_ref, u1a_ref, u1b_ref, ub1_ref, u2_ref,
              ub2_ref, out_ref):
    aggr = p0_ref[0] + p1_ref[0]
    t = x_ref[...] @ u1a_ref[...] + aggr @ u1b_ref[...] + ub1_ref[...]
    out_ref[...] = _silu(t) @ u2_ref[...] + ub2_ref[...]


def _final_mlp(x, partials, u1a, u1b, ub1, u2, ub2):
    return pl.pallas_call(
        _fin_body,
        grid=(N // NBLK_TC,),
        in_specs=[
            pl.BlockSpec((NBLK_TC, H), lambda i: (i, 0)),
            pl.BlockSpec((1, NBLK_TC, H), lambda i: (0, i, 0)),
            pl.BlockSpec((1, NBLK_TC, H), lambda i: (1, i, 0)),
            pl.BlockSpec((H, H), lambda i: (0, 0)),
            pl.BlockSpec((H, H), lambda i: (0, 0)),
            pl.BlockSpec((1, H), lambda i: (0, 0)),
            pl.BlockSpec((H, H), lambda i: (0, 0)),
            pl.BlockSpec((1, H), lambda i: (0, 0)),
        ],
        out_specs=pl.BlockSpec((NBLK_TC, H), lambda i: (i, 0)),
        out_shape=jax.ShapeDtypeStruct((N, H), jnp.float32),
    )(x, partials, partials, u1a, u1b, ub1, u2, ub2)


def kernel(x, pos, edge_index, W1, b1, W2, b2, U1, ub1, U2, ub2):
    send = jnp.pad(edge_index[0], (0, EPAD - E))
    rec_g = jnp.pad(edge_index[1], (0, EPAD - E))
    rec_s = jnp.pad(edge_index[1], (0, EPAD - E), constant_values=N)
    posp = jnp.pad(pos, ((0, 0), (0, PW - 3)))
    w1a = W1[:H]
    w1b = W1[H:2 * H]
    w1d = W1[2 * H:2 * H + 1]

    a_tab, b_tab = _build_tables(x, posp, w1a, w1b, b1[None, :])
    g, dmat = _sc_gather(a_tab, b_tab, send, rec_g)
    m = _edge_mlp(g, dmat, W2, w1d, b2[None, :])
    partials = _sc_scatter(m, rec_s)
    return _final_mlp(x, partials, U1[:H], U1[H:], ub1[None, :], U2,
                      ub2[None, :])


# speedup vs baseline: 1.3337x; 1.3095x over previous
import functools

import jax
import jax.numpy as jnp
from jax import lax
from jax.experimental import pallas as pl
from jax.experimental.pallas import tpu as pltpu
from jax.experimental.pallas import tpu_sc as plsc

N = 10000
E = 320000
H = 128
PW = 16
AW = H + PW

NC = 2
NS = 16
NW = NC * NS
K = 80
NBLK = 126
EPW = K * NBLK
EPAD = NW * EPW

NPAD = 10240
ROWS_PER_SUB = NPAD // NS
ZCH = K

EBLK = 1920
NBLK_TC = 1000


def _silu(v):
    return v * jax.nn.sigmoid(v)


def _pre_body(x_ref, posp_ref, w1a_ref, w1b_ref, b1_ref, a_ref, b_ref):
    xb = x_ref[...]
    pp = posp_ref[...]
    a_ref[:, :H] = xb @ w1a_ref[...] + b1_ref[...]
    a_ref[:, H:] = pp
    b_ref[:, :H] = xb @ w1b_ref[...]
    b_ref[:, H:] = -pp


def _build_tables(x, posp, w1a, w1b, b1):
    return pl.pallas_call(
        _pre_body,
        grid=(N // NBLK_TC,),
        in_specs=[
            pl.BlockSpec((NBLK_TC, H), lambda i: (i, 0)),
            pl.BlockSpec((NBLK_TC, PW), lambda i: (i, 0)),
            pl.BlockSpec((H, H), lambda i: (0, 0)),
            pl.BlockSpec((H, H), lambda i: (0, 0)),
            pl.BlockSpec((1, H), lambda i: (0, 0)),
        ],
        out_specs=[
            pl.BlockSpec((NBLK_TC, AW), lambda i: (i, 0)),
            pl.BlockSpec((NBLK_TC, AW), lambda i: (i, 0)),
        ],
        out_shape=[
            jax.ShapeDtypeStruct((N, AW), jnp.float32),
            jax.ShapeDtypeStruct((N, AW), jnp.float32),
        ],
    )(x, posp, w1a, w1b, b1)


_MESH = plsc.VectorSubcoreMesh(
    core_axis_name="c", subcore_axis_name="s", num_cores=NC, num_subcores=NS
)


@functools.partial(
    pl.kernel,
    out_type=[
        jax.ShapeDtypeStruct((EPAD, H), jnp.float32),
        jax.ShapeDtypeStruct((EPAD, PW), jnp.float32),
    ],
    mesh=_MESH,
    scratch_types=[
        pltpu.VMEM((K,), jnp.int32),
        pltpu.VMEM((K,), jnp.int32),
        pltpu.VMEM((K,), jnp.int32),
        pltpu.VMEM((K,), jnp.int32),
        pltpu.VMEM((K, AW), jnp.float32),
        pltpu.VMEM((K, AW), jnp.float32),
        pltpu.VMEM((K, AW), jnp.float32),
        pltpu.VMEM((K, AW), jnp.float32),
        pltpu.VMEM((K, H), jnp.float32),
        pltpu.VMEM((K, H), jnp.float32),
        pltpu.VMEM((K, PW), jnp.float32),
        pltpu.VMEM((K, PW), jnp.float32),
        pltpu.SemaphoreType.DMA,
        pltpu.SemaphoreType.DMA,
        pltpu.SemaphoreType.DMA,
        pltpu.SemaphoreType.DMA,
        pltpu.SemaphoreType.DMA,
        pltpu.SemaphoreType.DMA,
    ],
    compiler_params=pltpu.CompilerParams(use_tc_tiling_on_sc=False),
)
def _sc_gather(a_hbm, b_hbm, send_hbm, rec_hbm, g_hbm, d_hbm,
               s0, s1, r0, r1, ab0, ab1, bb0, bb1, gb0, gb1, db0, db1,
               si0, si1, sg0, sg1, sw0, sw1):
    cid = lax.axis_index("c")
    sid = lax.axis_index("s")
    wid = sid * NC + cid
    base0 = wid * EPW
    nblk = NBLK
    sidx = [s0, s1]
    ridx = [r0, r1]
    abuf = [ab0, ab1]
    bbuf = [bb0, bb1]
    gbuf = [gb0, gb1]
    dbuf = [db0, db1]
    isem = [si0, si1]
    gsem = [sg0, sg1]
    wsem = [sw0, sw1]

    def fire_idx(b, blk):
        base = base0 + blk * K
        pltpu.async_copy(send_hbm.at[pl.ds(base, K)], sidx[b], isem[b])
        pltpu.async_copy(rec_hbm.at[pl.ds(base, K)], ridx[b], isem[b])

    def wait_idx(b):
        pltpu.make_async_copy(send_hbm.at[pl.ds(0, K)], sidx[b], isem[b]).wait()
        pltpu.make_async_copy(rec_hbm.at[pl.ds(0, K)], ridx[b], isem[b]).wait()

    def fire_gather(b):
        pltpu.async_copy(a_hbm.at[sidx[b]], abuf[b], gsem[b])
        pltpu.async_copy(b_hbm.at[ridx[b]], bbuf[b], gsem[b])

    def wait_gather(b):
        pltpu.make_async_copy(a_hbm.at[sidx[b]], abuf[b], gsem[b]).wait()
        pltpu.make_async_copy(b_hbm.at[ridx[b]], bbuf[b], gsem[b]).wait()

    def fire_write(b, blk):
        base = base0 + blk * K
        pltpu.async_copy(gbuf[b], g_hbm.at[pl.ds(base, K)], wsem[b])
        pltpu.async_copy(dbuf[b], d_hbm.at[pl.ds(base, K)], wsem[b])

    def wait_write(b):
        pltpu.make_async_copy(gbuf[b], g_hbm.at[pl.ds(0, K)], wsem[b]).wait()
        pltpu.make_async_copy(dbuf[b], d_hbm.at[pl.ds(0, K)], wsem[b]).wait()

    def compute(b):
        ab = abuf[b]
        bb = bbuf[b]
        gb = gbuf[b]
        db = dbuf[b]

        @plsc.parallel_loop(0, K, unroll=2)
        def row(j):
            for q in range(H // 16):
                sl = pl.ds(q * 16, 16)
                gb[j, sl] = ab[j, sl] + bb[j, sl]
            db[j, :] = ab[j, pl.ds(H, PW)] + bb[j, pl.ds(H, PW)]

    fire_idx(0, 0)
    wait_idx(0)
    fire_gather(0)
    fire_idx(1, 1)

    def pair(ii, carry):
        for b in range(2):
            i = ii * 2 + b
            nb = 1 - b
            wait_gather(b)

            @pl.when(i + 2 < nblk)
            def _():
                fire_idx(b, i + 2)

            @pl.when(i + 1 < nblk)
            def _():
                wait_idx(nb)
                fire_gather(nb)

            @pl.when(i >= 2)
            def _():
                wait_write(b)

            compute(b)
            fire_write(b, i)
        return carry

    lax.fori_loop(0, nblk // 2, pair, 0)
    wait_write(0)
    wait_write(1)


def _edge_body(g_ref, d_ref, w2_ref, wd_ref, b2_ref, m_ref):
    d = d_ref[...]
    dist = jnp.sqrt(jnp.sum(d * d, axis=1, keepdims=True))
    h = _silu(g_ref[...] + dist * wd_ref[...])
    m_ref[...] = _silu(h @ w2_ref[...] + b2_ref[...])


def _edge_mlp(g, dmat, w2, w1d, b2):
    return pl.pallas_call(
        _edge_body,
        grid=(EPAD // EBLK,),
        in_specs=[
            pl.BlockSpec((EBLK, H), lambda i: (i, 0)),
            pl.BlockSpec((EBLK, PW), lambda i: (i, 0)),
            pl.BlockSpec((H, H), lambda i: (0, 0)),
            pl.BlockSpec((1, H), lambda i: (0, 0)),
            pl.BlockSpec((1, H), lambda i: (0, 0)),
        ],
        out_specs=pl.BlockSpec((EBLK, H), lambda i: (i, 0)),
        out_shape=jax.ShapeDtypeStruct((EPAD, H), jnp.float32),
    )(g, dmat, w2, w1d, b2)


@functools.partial(
    pl.kernel,
    out_type=jax.ShapeDtypeStruct((NC, NPAD, H), jnp.float32),
    mesh=_MESH,
    scratch_types=[
        pltpu.VMEM((K,), jnp.int32),
        pltpu.VMEM((K,), jnp.int32),
        pltpu.VMEM((K, H), jnp.float32),
        pltpu.VMEM((K, H), jnp.float32),
        pltpu.VMEM_SHARED((NPAD, H), jnp.float32),
        pltpu.SemaphoreType.DMA,
        pltpu.SemaphoreType.DMA,
        pltpu.SemaphoreType.DMA,
        pltpu.SemaphoreType.DMA,
    ],
    compiler_params=pltpu.CompilerParams(use_tc_tiling_on_sc=False),
)
def _sc_scatter(m_hbm, rec_hbm, out_hbm, r0, r1, mb0, mb1, aggr,
                sl0, sl1, sa0, sa1):
    cid = lax.axis_index("c")
    sid = lax.axis_index("s")
    wid = sid * NC + cid
    base0 = wid * EPW
    ridx = [r0, r1]
    mbuf = [mb0, mb1]
    lsem = [sl0, sl1]
    asem = [sa0, sa1]

    def zrow(j, c2):
        for q in range(H // 16):
            mb0[j, pl.ds(q * 16, 16)] = jnp.zeros((16,), jnp.float32)
        return c2

    lax.fori_loop(0, ZCH, zrow, 0)

    def zchunk(t, c2):
        pltpu.sync_copy(mb0, aggr.at[pl.ds(sid * ROWS_PER_SUB + t * ZCH, ZCH)])
        return c2

    lax.fori_loop(0, ROWS_PER_SUB // ZCH, zchunk, 0)
    plsc.subcore_barrier()

    def fire_load(b, blk):
        base = base0 + blk * K
        pltpu.async_copy(rec_hbm.at[pl.ds(base, K)], ridx[b], lsem[b])
        pltpu.async_copy(m_hbm.at[pl.ds(base, K)], mbuf[b], lsem[b])

    def wait_load(b):
        pltpu.make_async_copy(rec_hbm.at[pl.ds(0, K)], ridx[b], lsem[b]).wait()
        pltpu.make_async_copy(m_hbm.at[pl.ds(0, K)], mbuf[b], lsem[b]).wait()

    def fire_add(b):
        pltpu.async_copy(mbuf[b], aggr.at[ridx[b]], asem[b], add=True)

    def wait_add(b):
        pltpu.make_async_copy(mbuf[b], aggr.at[ridx[b]], asem[b]).wait()

    fire_load(0, 0)
    fire_load(1, 1)

    def pair(ii, carry):
        for b in range(2):
            i = ii * 2 + b
            wait_load(b)
            fire_add(b)

            @pl.when(i + 2 < NBLK)
            def _():
                wait_add(b)
                fire_load(b, i + 2)

        return carry

    lax.fori_loop(0, NBLK // 2, pair, 0)
    wait_add(0)
    wait_add(1)
    plsc.subcore_barrier()

    def dchunk(t, c2):
        row0 = sid * ROWS_PER_SUB + t * ZCH
        pltpu.sync_copy(aggr.at[pl.ds(row0, ZCH)], mb0)
        pltpu.sync_copy(mb0, out_hbm.at[cid, pl.ds(row0, ZCH)])
        return c2

    lax.fori_loop(0, ROWS_PER_SUB // ZCH, dchunk, 0)


def _fin_body(x_ref, p0_ref, p1_ref, u1a_ref, u1b_ref, ub1_ref, u2_ref,
              ub2_ref, out_ref):
    aggr = p0_ref[0] + p1_ref[0]
    t = x_ref[...] @ u1a_ref[...] + aggr @ u1b_ref[...] + ub1_ref[...]
    out_ref[...] = _silu(t) @ u2_ref[...] + ub2_ref[...]


def _final_mlp(x, partials, u1a, u1b, ub1, u2, ub2):
    return pl.pallas_call(
        _fin_body,
        grid=(N // NBLK_TC,),
        in_specs=[
            pl.BlockSpec((NBLK_TC, H), lambda i: (i, 0)),
            pl.BlockSpec((1, NBLK_TC, H), lambda i: (0, i, 0)),
            pl.BlockSpec((1, NBLK_TC, H), lambda i: (1, i, 0)),
            pl.BlockSpec((H, H), lambda i: (0, 0)),
            pl.BlockSpec((H, H), lambda i: (0, 0)),
            pl.BlockSpec((1, H), lambda i: (0, 0)),
            pl.BlockSpec((H, H), lambda i: (0, 0)),
            pl.BlockSpec((1, H), lambda i: (0, 0)),
        ],
        out_specs=pl.BlockSpec((NBLK_TC, H), lambda i: (i, 0)),
        out_shape=jax.ShapeDtypeStruct((N, H), jnp.float32),
    )(x, partials, partials, u1a, u1b, ub1, u2, ub2)


def kernel(x, pos, edge_index, W1, b1, W2, b2, U1, ub1, U2, ub2):
    send = jnp.pad(edge_index[0], (0, EPAD - E))
    rec_g = jnp.pad(edge_index[1], (0, EPAD - E))
    rec_s = jnp.pad(edge_index[1], (0, EPAD - E), constant_values=N)
    posp = jnp.pad(pos, ((0, 0), (0, PW - 3)))
    w1a = W1[:H]
    w1b = W1[H:2 * H]
    w1d = W1[2 * H:2 * H + 1]

    a_tab, b_tab = _build_tables(x, posp, w1a, w1b, b1[None, :])
    g, dmat = _sc_gather(a_tab, b_tab, send, rec_g)
    m = _edge_mlp(g, dmat, W2, w1d, b2[None, :])
    partials = _sc_scatter(m, rec_s)
    return _final_mlp(x, partials, U1[:H], U1[H:], ub1[None, :], U2,
                      ub2[None, :])
